# Initial kernel scaffold; baseline (speedup 1.0000x reference)
import functools

import jax
import jax.numpy as jnp
from jax import lax
from jax.experimental import pallas as pl
from jax.experimental.pallas import tpu as pltpu
from jax.experimental.pallas import tpu_sc as plsc

N_NODES = 10000
N_EDGES = 320000
D_EDGE = 16
D_NODE = 128
D_GLOBAL = 128
LATENT = 32
D_OUT = 128

NC = 2
NS = 16
NW = NC * NS
E_PER_W = N_EDGES // NW
CHUNK = 125
CHUNKS = E_PER_W // CHUNK
ROWS_PER_TILE = N_NODES // NS


def _sc_segment_sum(edge_attr, idx3):
    mesh = plsc.VectorSubcoreMesh(core_axis_name="c", subcore_axis_name="s")

    @functools.partial(
        pl.kernel,
        mesh=mesh,
        out_type=jax.ShapeDtypeStruct((NC, N_NODES, D_EDGE), jnp.float32),
        scratch_types=[
            pltpu.VMEM((CHUNKS, CHUNK), jnp.int32),
            pltpu.VMEM((CHUNK, D_EDGE), jnp.float32),
            pltpu.VMEM((ROWS_PER_TILE, D_EDGE), jnp.float32),
            pltpu.VMEM_SHARED((N_NODES, D_EDGE), jnp.float32),
        ],
    )
    def seg_sum(edge_hbm, idx_hbm, out_hbm, idx_v, ebuf, zbuf, agg_sh):
        cid = lax.axis_index("c")
        sid = lax.axis_index("s")
        wid = cid * NS + sid

        def zero_row(i, carry):
            zbuf[i, :] = jnp.zeros((16,), jnp.float32)
            return carry
        lax.fori_loop(0, ROWS_PER_TILE, zero_row, 0)
        row0 = sid * ROWS_PER_TILE
        pltpu.sync_copy(zbuf, agg_sh.at[pl.ds(row0, ROWS_PER_TILE)])
        plsc.subcore_barrier()

        pltpu.sync_copy(idx_hbm.at[wid], idx_v)

        base = wid * E_PER_W

        def chunk_body(c, carry):
            pltpu.sync_copy(edge_hbm.at[pl.ds(base + c * CHUNK, CHUNK)], ebuf)
            pltpu.sync_copy(ebuf, agg_sh.at[idx_v.at[c]], add=True)
            return carry
        lax.fori_loop(0, CHUNKS, chunk_body, 0)

        plsc.subcore_barrier()
        pltpu.sync_copy(agg_sh.at[pl.ds(row0, ROWS_PER_TILE)],
                        out_hbm.at[cid, pl.ds(row0, ROWS_PER_TILE)])

    return seg_sum(edge_attr, idx3)


BLK = 2000


def _tc_mlp_body(x_ref, p_ref, g_ref, w1x_ref, w1e_ref, w1g_ref, b1_ref,
                 w2_ref, b2_ref, o_ref):
    bias = (jnp.dot(g_ref[...], w1g_ref[...],
                    preferred_element_type=jnp.float32) + b1_ref[...])
    agg = p_ref[0] + p_ref[1]
    h = jnp.dot(x_ref[...], w1x_ref[...], preferred_element_type=jnp.float32)
    h = h + jnp.dot(agg, w1e_ref[...], preferred_element_type=jnp.float32)
    h = jnp.maximum(h + bias, 0.0)
    o_ref[...] = (jnp.dot(h, w2_ref[...], preferred_element_type=jnp.float32)
                  + b2_ref[...])


def _tc_mlp(x, partials, global_attr, w1x, w1e, w1g, b1, w2, b2):
    grid = (N_NODES // BLK,)
    return pl.pallas_call(
        _tc_mlp_body,
        grid=grid,
        in_specs=[
            pl.BlockSpec((BLK, D_NODE), lambda i: (i, 0)),
            pl.BlockSpec((NC, BLK, D_EDGE), lambda i: (0, i, 0)),
            pl.BlockSpec((1, D_GLOBAL), lambda i: (0, 0)),
            pl.BlockSpec((D_NODE, LATENT), lambda i: (0, 0)),
            pl.BlockSpec((D_EDGE, LATENT), lambda i: (0, 0)),
            pl.BlockSpec((D_GLOBAL, LATENT), lambda i: (0, 0)),
            pl.BlockSpec((1, LATENT), lambda i: (0, 0)),
            pl.BlockSpec((LATENT, D_OUT), lambda i: (0, 0)),
            pl.BlockSpec((1, D_OUT), lambda i: (0, 0)),
        ],
        out_specs=pl.BlockSpec((BLK, D_OUT), lambda i: (i, 0)),
        out_shape=jax.ShapeDtypeStruct((N_NODES, D_OUT), jnp.float32),
    )(x, partials, global_attr, w1x, w1e, w1g, b1, w2, b2)


def kernel(x, edge_index, edge_attr, global_attr, W1, b1, W2, b2):
    recv = edge_index[1].astype(jnp.int32)
    idx3 = recv.reshape(NW, CHUNKS, CHUNK)
    partials = _sc_segment_sum(edge_attr, idx3)
    w1x = W1[:D_NODE]
    w1e = W1[D_NODE:D_NODE + D_EDGE]
    w1g = W1[D_NODE + D_EDGE:]
    return _tc_mlp(x, partials, global_attr, w1x, w1e, w1g,
                   b1.reshape(1, LATENT), W2, b2.reshape(1, D_OUT))


# speedup vs baseline: 3.2664x; 3.2664x over previous
import functools

import jax
import jax.numpy as jnp
from jax import lax
from jax.experimental import pallas as pl
from jax.experimental.pallas import tpu as pltpu
from jax.experimental.pallas import tpu_sc as plsc

N_NODES = 10000
N_EDGES = 320000
D_EDGE = 16
D_NODE = 128
D_GLOBAL = 128
LATENT = 32
D_OUT = 128

NC = 2
NS = 16
NW = NC * NS
E_PER_W = N_EDGES // NW
CHUNK = 80
CHUNKS = E_PER_W // CHUNK
N_NODES_PAD = 10240
ROWS_PER_TILE = N_NODES_PAD // NS


def _sc_segment_sum(edge_attr, recv, zeros_pad):
    mesh = plsc.VectorSubcoreMesh(core_axis_name="c", subcore_axis_name="s")

    @functools.partial(
        pl.kernel,
        mesh=mesh,
        compiler_params=pltpu.CompilerParams(use_tc_tiling_on_sc=False),
        out_type=jax.ShapeDtypeStruct((NC, N_NODES_PAD, D_EDGE), jnp.float32),
        scratch_types=[
            pltpu.VMEM((CHUNK,), jnp.int32),
            pltpu.VMEM((CHUNK, D_EDGE), jnp.float32),
            pltpu.VMEM_SHARED((N_NODES_PAD, D_EDGE), jnp.float32),
        ],
    )
    def seg_sum(edge_hbm, idx_hbm, zeros_hbm, out_hbm, idx_v, ebuf, agg_sh):
        cid = lax.axis_index("c")
        sid = lax.axis_index("s")
        wid = cid * NS + sid
        row0 = sid * ROWS_PER_TILE

        pltpu.sync_copy(zeros_hbm.at[pl.ds(row0, ROWS_PER_TILE)],
                        agg_sh.at[pl.ds(row0, ROWS_PER_TILE)])
        plsc.subcore_barrier()

        base = wid * E_PER_W

        def chunk_body(c, carry):
            off = base + c * CHUNK
            pltpu.sync_copy(idx_hbm.at[pl.ds(off, CHUNK)], idx_v)
            pltpu.sync_copy(edge_hbm.at[pl.ds(off, CHUNK)], ebuf)
            pltpu.sync_copy(ebuf, agg_sh.at[idx_v], add=True)
            return carry
        lax.fori_loop(0, CHUNKS, chunk_body, 0)

        plsc.subcore_barrier()
        pltpu.sync_copy(agg_sh.at[pl.ds(row0, ROWS_PER_TILE)],
                        out_hbm.at[cid, pl.ds(row0, ROWS_PER_TILE)])

    return seg_sum(edge_attr, recv, zeros_pad)


BLK = 2000


def _tc_mlp_body(x_ref, p_ref, g_ref, w1x_ref, w1e_ref, w1g_ref, b1_ref,
                 w2_ref, b2_ref, o_ref):
    bias = (jnp.dot(g_ref[...], w1g_ref[...],
                    preferred_element_type=jnp.float32) + b1_ref[...])
    agg = p_ref[0] + p_ref[1]
    h = jnp.dot(x_ref[...], w1x_ref[...], preferred_element_type=jnp.float32)
    h = h + jnp.dot(agg, w1e_ref[...], preferred_element_type=jnp.float32)
    h = jnp.maximum(h + bias, 0.0)
    o_ref[...] = (jnp.dot(h, w2_ref[...], preferred_element_type=jnp.float32)
                  + b2_ref[...])


def _tc_mlp(x, partials, global_attr, w1x, w1e, w1g, b1, w2, b2):
    grid = (N_NODES // BLK,)
    return pl.pallas_call(
        _tc_mlp_body,
        grid=grid,
        in_specs=[
            pl.BlockSpec((BLK, D_NODE), lambda i: (i, 0)),
            pl.BlockSpec((NC, BLK, D_EDGE), lambda i: (0, i, 0)),
            pl.BlockSpec((1, D_GLOBAL), lambda i: (0, 0)),
            pl.BlockSpec((D_NODE, LATENT), lambda i: (0, 0)),
            pl.BlockSpec((D_EDGE, LATENT), lambda i: (0, 0)),
            pl.BlockSpec((D_GLOBAL, LATENT), lambda i: (0, 0)),
            pl.BlockSpec((1, LATENT), lambda i: (0, 0)),
            pl.BlockSpec((LATENT, D_OUT), lambda i: (0, 0)),
            pl.BlockSpec((1, D_OUT), lambda i: (0, 0)),
        ],
        out_specs=pl.BlockSpec((BLK, D_OUT), lambda i: (i, 0)),
        out_shape=jax.ShapeDtypeStruct((N_NODES, D_OUT), jnp.float32),
    )(x, partials, global_attr, w1x, w1e, w1g, b1, w2, b2)


def kernel(x, edge_index, edge_attr, global_attr, W1, b1, W2, b2):
    recv = edge_index[1].astype(jnp.int32)
    zeros_pad = jnp.zeros((N_NODES_PAD, D_EDGE), jnp.float32)
    partials = _sc_segment_sum(edge_attr, recv, zeros_pad)
    w1x = W1[:D_NODE]
    w1e = W1[D_NODE:D_NODE + D_EDGE]
    w1g = W1[D_NODE + D_EDGE:]
    return _tc_mlp(x, partials, global_attr, w1x, w1e, w1g,
                   b1.reshape(1, LATENT), W2, b2.reshape(1, D_OUT))


# speedup vs baseline: 5.6872x; 1.7411x over previous
import functools

import jax
import jax.numpy as jnp
from jax import lax
from jax.experimental import pallas as pl
from jax.experimental.pallas import tpu as pltpu
from jax.experimental.pallas import tpu_sc as plsc

N_NODES = 10000
N_EDGES = 320000
D_EDGE = 16
D_NODE = 128
D_GLOBAL = 128
LATENT = 32
D_OUT = 128

NC = 2
NS = 16
NW = NC * NS
E_PER_W = N_EDGES // NW
CHUNK = 2000
CHUNKS = E_PER_W // CHUNK
N_NODES_PAD = 10240
ROWS_PER_TILE = N_NODES_PAD // NS


def _sc_segment_sum(edge_attr, recv, zeros_pad):
    mesh = plsc.VectorSubcoreMesh(core_axis_name="c", subcore_axis_name="s")

    @functools.partial(
        pl.kernel,
        mesh=mesh,
        compiler_params=pltpu.CompilerParams(use_tc_tiling_on_sc=False),
        out_type=jax.ShapeDtypeStruct((NC, N_NODES_PAD, D_EDGE), jnp.float32),
        scratch_types=[
            pltpu.VMEM((2, CHUNK), jnp.int32),
            pltpu.VMEM((2, CHUNK, D_EDGE), jnp.float32),
            pltpu.VMEM_SHARED((N_NODES_PAD, D_EDGE), jnp.float32),
            pltpu.SemaphoreType.DMA,
            pltpu.SemaphoreType.DMA,
        ],
    )
    def seg_sum(edge_hbm, idx_hbm, zeros_hbm, out_hbm, idx_v, ebuf, agg_sh,
                lsem0, lsem1):
        cid = lax.axis_index("c")
        sid = lax.axis_index("s")
        wid = cid * NS + sid
        row0 = sid * ROWS_PER_TILE
        base = wid * E_PER_W
        lsems = (lsem0, lsem1)

        def start_load(j):
            b = j % 2
            off = base + j * CHUNK
            ci = pltpu.async_copy(idx_hbm.at[pl.ds(off, CHUNK)], idx_v.at[b],
                                  lsems[b])
            ce = pltpu.async_copy(edge_hbm.at[pl.ds(off, CHUNK)], ebuf.at[b],
                                  lsems[b])
            return ci, ce

        pend = start_load(0)
        pltpu.sync_copy(zeros_hbm.at[pl.ds(row0, ROWS_PER_TILE)],
                        agg_sh.at[pl.ds(row0, ROWS_PER_TILE)])
        plsc.subcore_barrier()

        for j in range(CHUNKS):
            ci, ce = pend
            ci.wait()
            ce.wait()
            if j + 1 < CHUNKS:
                pend = start_load(j + 1)
            b = j % 2
            pltpu.sync_copy(ebuf.at[b], agg_sh.at[idx_v.at[b]], add=True)

        plsc.subcore_barrier()
        pltpu.sync_copy(agg_sh.at[pl.ds(row0, ROWS_PER_TILE)],
                        out_hbm.at[cid, pl.ds(row0, ROWS_PER_TILE)])

    return seg_sum(edge_attr, recv, zeros_pad)


BLK = 2000


def _tc_mlp_body(x_ref, p_ref, g_ref, w1x_ref, w1e_ref, w1g_ref, b1_ref,
                 w2_ref, b2_ref, o_ref):
    bias = (jnp.dot(g_ref[...], w1g_ref[...],
                    preferred_element_type=jnp.float32) + b1_ref[...])
    agg = p_ref[0] + p_ref[1]
    h = jnp.dot(x_ref[...], w1x_ref[...], preferred_element_type=jnp.float32)
    h = h + jnp.dot(agg, w1e_ref[...], preferred_element_type=jnp.float32)
    h = jnp.maximum(h + bias, 0.0)
    o_ref[...] = (jnp.dot(h, w2_ref[...], preferred_element_type=jnp.float32)
                  + b2_ref[...])


def _tc_mlp(x, partials, global_attr, w1x, w1e, w1g, b1, w2, b2):
    grid = (N_NODES // BLK,)
    return pl.pallas_call(
        _tc_mlp_body,
        grid=grid,
        in_specs=[
            pl.BlockSpec((BLK, D_NODE), lambda i: (i, 0)),
            pl.BlockSpec((NC, BLK, D_EDGE), lambda i: (0, i, 0)),
            pl.BlockSpec((1, D_GLOBAL), lambda i: (0, 0)),
            pl.BlockSpec((D_NODE, LATENT), lambda i: (0, 0)),
            pl.BlockSpec((D_EDGE, LATENT), lambda i: (0, 0)),
            pl.BlockSpec((D_GLOBAL, LATENT), lambda i: (0, 0)),
            pl.BlockSpec((1, LATENT), lambda i: (0, 0)),
            pl.BlockSpec((LATENT, D_OUT), lambda i: (0, 0)),
            pl.BlockSpec((1, D_OUT), lambda i: (0, 0)),
        ],
        out_specs=pl.BlockSpec((BLK, D_OUT), lambda i: (i, 0)),
        out_shape=jax.ShapeDtypeStruct((N_NODES, D_OUT), jnp.float32),
    )(x, partials, global_attr, w1x, w1e, w1g, b1, w2, b2)


def kernel(x, edge_index, edge_attr, global_attr, W1, b1, W2, b2):
    recv = edge_index[1].astype(jnp.int32)
    zeros_pad = jnp.zeros((N_NODES_PAD, D_EDGE), jnp.float32)
    partials = _sc_segment_sum(edge_attr, recv, zeros_pad)
    w1x = W1[:D_NODE]
    w1e = W1[D_NODE:D_NODE + D_EDGE]
    w1g = W1[D_NODE + D_EDGE:]
    return _tc_mlp(x, partials, global_attr, w1x, w1e, w1g,
                   b1.reshape(1, LATENT), W2, b2.reshape(1, D_OUT))


# speedup vs baseline: 9.2280x; 1.6226x over previous
import functools

import jax
import jax.numpy as jnp
from jax import lax
from jax.experimental import pallas as pl
from jax.experimental.pallas import tpu as pltpu
from jax.experimental.pallas import tpu_sc as plsc

N_NODES = 10000
N_EDGES = 320000
D_EDGE = 16
D_NODE = 128
D_GLOBAL = 128
LATENT = 32
D_OUT = 128

NC = 2
NS = 16
TR = 2
SL = 8
LN = 128
TCOLS = N_EDGES // LN
TC_PER_CORE = TCOLS // NC
CC = 125
CHUNKS = TC_PER_CORE // CC
E_CHUNK = CC * LN
N_NODES_PAD = 10240


def _sc_segment_sum(ea4, recv):
    mesh = plsc.VectorSubcoreMesh(core_axis_name="c", subcore_axis_name="s")

    @functools.partial(
        pl.kernel,
        mesh=mesh,
        compiler_params=pltpu.CompilerParams(
            use_tc_tiling_on_sc=False, needs_layout_passes=False),
        out_type=jax.ShapeDtypeStruct((NC, D_EDGE, N_NODES_PAD), jnp.float32),
        scratch_types=[
            pltpu.VMEM((2, CC, LN), jnp.float32),
            pltpu.VMEM((2, E_CHUNK), jnp.int32),
            pltpu.VMEM((N_NODES_PAD,), jnp.float32),
            pltpu.SemaphoreType.DMA,
            pltpu.SemaphoreType.DMA,
        ],
    )
    def seg_sum(ea_hbm, idx_hbm, out_hbm, ebuf, ibuf, acc, sem0, sem1):
        cid = lax.axis_index("c")
        sid = lax.axis_index("s")
        tr = sid // SL
        s = sid % SL
        tc0 = cid * TC_PER_CORE
        e0 = cid * (N_EDGES // NC)
        sems = (sem0, sem1)

        def start_load(j):
            b = j % 2
            cv = pltpu.async_copy(
                ea_hbm.at[tr, pl.ds(tc0 + j * CC, CC), s, :], ebuf.at[b],
                sems[b])
            ci = pltpu.async_copy(
                idx_hbm.at[pl.ds(e0 + j * E_CHUNK, E_CHUNK)], ibuf.at[b],
                sems[b])
            return cv, ci

        pend = start_load(0)

        def zero(i, carry):
            acc[pl.ds(i * 16, 16)] = jnp.zeros((16,), jnp.float32)
            return carry
        lax.fori_loop(0, N_NODES_PAD // 16, zero, 0)

        for j in range(CHUNKS):
            cv, ci = pend
            cv.wait()
            ci.wait()
            if j + 1 < CHUNKS:
                pend = start_load(j + 1)
            b = j % 2

            def row_body(r, carry):
                for g in range(SL):
                    idx = ibuf[b, pl.ds(r * LN + g * 16, 16)]
                    v = ebuf[b, r, pl.ds(g * 16, 16)]
                    plsc.addupdate_scatter(acc, [idx], v)
                return carry
            lax.fori_loop(0, CC, row_body, 0)

        pltpu.sync_copy(acc, out_hbm.at[cid, sid])

    return seg_sum(ea4, recv)


BLK = 2048


def _tc_mlp_body(x_ref, p_ref, g_ref, w1x_ref, w1e_ref, w1g_ref, b1_ref,
                 w2_ref, b2_ref, o_ref):
    bias = (jnp.dot(g_ref[...], w1g_ref[...],
                    preferred_element_type=jnp.float32) + b1_ref[...])
    agg_t = p_ref[0] + p_ref[1]
    h = jnp.dot(x_ref[...], w1x_ref[...], preferred_element_type=jnp.float32)
    h = h + lax.dot_general(agg_t, w1e_ref[...], (((0,), (0,)), ((), ())),
                            preferred_element_type=jnp.float32)
    h = jnp.maximum(h + bias, 0.0)
    o_ref[...] = (jnp.dot(h, w2_ref[...], preferred_element_type=jnp.float32)
                  + b2_ref[...])


def _tc_mlp(x, partials, global_attr, w1x, w1e, w1g, b1, w2, b2):
    grid = (N_NODES_PAD // BLK,)
    return pl.pallas_call(
        _tc_mlp_body,
        grid=grid,
        in_specs=[
            pl.BlockSpec((BLK, D_NODE), lambda i: (i, 0)),
            pl.BlockSpec((NC, D_EDGE, BLK), lambda i: (0, 0, i)),
            pl.BlockSpec((1, D_GLOBAL), lambda i: (0, 0)),
            pl.BlockSpec((D_NODE, LATENT), lambda i: (0, 0)),
            pl.BlockSpec((D_EDGE, LATENT), lambda i: (0, 0)),
            pl.BlockSpec((D_GLOBAL, LATENT), lambda i: (0, 0)),
            pl.BlockSpec((1, LATENT), lambda i: (0, 0)),
            pl.BlockSpec((LATENT, D_OUT), lambda i: (0, 0)),
            pl.BlockSpec((1, D_OUT), lambda i: (0, 0)),
        ],
        out_specs=pl.BlockSpec((BLK, D_OUT), lambda i: (i, 0)),
        out_shape=jax.ShapeDtypeStruct((N_NODES, D_OUT), jnp.float32),
    )(x, partials, global_attr, w1x, w1e, w1g, b1, w2, b2)


def kernel(x, edge_index, edge_attr, global_attr, W1, b1, W2, b2):
    recv = edge_index[1].astype(jnp.int32)
    ea4 = edge_attr.T.reshape(TR, SL, TCOLS, LN).transpose(0, 2, 1, 3)
    partials = _sc_segment_sum(ea4, recv)
    w1x = W1[:D_NODE]
    w1e = W1[D_NODE:D_NODE + D_EDGE]
    w1g = W1[D_NODE + D_EDGE:]
    return _tc_mlp(x, partials, global_attr, w1x, w1e, w1g,
                   b1.reshape(1, LATENT), W2, b2.reshape(1, D_OUT))


# speedup vs baseline: 10.0398x; 1.0880x over previous
import functools

import jax
import jax.numpy as jnp
from jax import lax
from jax.experimental import pallas as pl
from jax.experimental.pallas import tpu as pltpu
from jax.experimental.pallas import tpu_sc as plsc

N_NODES = 10000
N_EDGES = 320000
D_EDGE = 16
D_NODE = 128
D_GLOBAL = 128
LATENT = 32
D_OUT = 128

NC = 2
NS = 16
TR = 2
SL = 8
LN = 128
TCOLS = N_EDGES // LN
TC_PER_CORE = TCOLS // NC
CC = 125
CHUNKS = TC_PER_CORE // CC
E_CHUNK = CC * LN
N_NODES_PAD = 10240


def _sc_segment_sum(ea4, ei3):
    mesh = plsc.VectorSubcoreMesh(core_axis_name="c", subcore_axis_name="s")

    @functools.partial(
        pl.kernel,
        mesh=mesh,
        compiler_params=pltpu.CompilerParams(
            use_tc_tiling_on_sc=False, needs_layout_passes=False),
        out_type=jax.ShapeDtypeStruct((NC, D_EDGE, N_NODES_PAD), jnp.float32),
        scratch_types=[
            pltpu.VMEM((2, CC, LN), jnp.float32),
            pltpu.VMEM((2, CC, LN), jnp.int32),
            pltpu.VMEM((N_NODES_PAD,), jnp.float32),
            pltpu.VMEM((N_NODES_PAD,), jnp.float32),
            pltpu.SemaphoreType.DMA,
            pltpu.SemaphoreType.DMA,
        ],
    )
    def seg_sum(ea_hbm, ei_hbm, out_hbm, ebuf, ibuf, acc_a, acc_b,
                sem0, sem1):
        cid = lax.axis_index("c")
        sid = lax.axis_index("s")
        tr = sid // SL
        s = sid % SL
        tc0 = cid * TC_PER_CORE
        sems = (sem0, sem1)

        def start_load(j):
            b = j % 2
            cv = pltpu.async_copy(
                ea_hbm.at[tr, pl.ds(tc0 + j * CC, CC), s, :], ebuf.at[b],
                sems[b])
            ci = pltpu.async_copy(
                ei_hbm.at[pl.ds(tc0 + j * CC, CC), 1, :], ibuf.at[b],
                sems[b])
            return cv, ci

        pend = start_load(0)

        zv = jnp.zeros((16,), jnp.float32)

        def zero(i, carry):
            acc_a[pl.ds(i * 16, 16)] = zv
            acc_b[pl.ds(i * 16, 16)] = zv
            return carry
        lax.fori_loop(0, N_NODES_PAD // 16, zero, 0)

        for j in range(CHUNKS):
            cv, ci = pend
            cv.wait()
            ci.wait()
            if j + 1 < CHUNKS:
                pend = start_load(j + 1)
            b = j % 2

            def row_body(r, carry):
                for g in range(SL):
                    idx = ibuf[b, r, pl.ds(g * 16, 16)]
                    v = ebuf[b, r, pl.ds(g * 16, 16)]
                    plsc.addupdate_scatter(acc_a if g % 2 == 0 else acc_b,
                                           [idx], v)
                return carry
            lax.fori_loop(0, CC, row_body, 0)

        def merge(i, carry):
            sl = pl.ds(i * 16, 16)
            acc_a[sl] = acc_a[sl] + acc_b[sl]
            return carry
        lax.fori_loop(0, N_NODES_PAD // 16, merge, 0)

        pltpu.sync_copy(acc_a, out_hbm.at[cid, sid])

    return seg_sum(ea4, ei3)


BLK = 2048


def _tc_mlp_body(x_ref, p_ref, g_ref, w1x_ref, w1e_ref, w1g_ref, b1_ref,
                 w2_ref, b2_ref, o_ref):
    bias = (jnp.dot(g_ref[...], w1g_ref[...],
                    preferred_element_type=jnp.float32) + b1_ref[...])
    agg_t = p_ref[0] + p_ref[1]
    h = jnp.dot(x_ref[...], w1x_ref[...], preferred_element_type=jnp.float32)
    h = h + lax.dot_general(agg_t, w1e_ref[...], (((0,), (0,)), ((), ())),
                            preferred_element_type=jnp.float32)
    h = jnp.maximum(h + bias, 0.0)
    o_ref[...] = (jnp.dot(h, w2_ref[...], preferred_element_type=jnp.float32)
                  + b2_ref[...])


def _tc_mlp(x, partials, global_attr, w1x, w1e, w1g, b1, w2, b2):
    grid = (N_NODES_PAD // BLK,)
    return pl.pallas_call(
        _tc_mlp_body,
        grid=grid,
        in_specs=[
            pl.BlockSpec((BLK, D_NODE), lambda i: (i, 0)),
            pl.BlockSpec((NC, D_EDGE, BLK), lambda i: (0, 0, i)),
            pl.BlockSpec((1, D_GLOBAL), lambda i: (0, 0)),
            pl.BlockSpec((D_NODE, LATENT), lambda i: (0, 0)),
            pl.BlockSpec((D_EDGE, LATENT), lambda i: (0, 0)),
            pl.BlockSpec((D_GLOBAL, LATENT), lambda i: (0, 0)),
            pl.BlockSpec((1, LATENT), lambda i: (0, 0)),
            pl.BlockSpec((LATENT, D_OUT), lambda i: (0, 0)),
            pl.BlockSpec((1, D_OUT), lambda i: (0, 0)),
        ],
        out_specs=pl.BlockSpec((BLK, D_OUT), lambda i: (i, 0)),
        out_shape=jax.ShapeDtypeStruct((N_NODES, D_OUT), jnp.float32),
    )(x, partials, global_attr, w1x, w1e, w1g, b1, w2, b2)


def kernel(x, edge_index, edge_attr, global_attr, W1, b1, W2, b2):
    ea4 = edge_attr.T.reshape(TR, SL, TCOLS, LN).transpose(0, 2, 1, 3)
    ei3 = edge_index.astype(jnp.int32).reshape(2, TCOLS, LN).transpose(1, 0, 2)
    partials = _sc_segment_sum(ea4, ei3)
    w1x = W1[:D_NODE]
    w1e = W1[D_NODE:D_NODE + D_EDGE]
    w1g = W1[D_NODE + D_EDGE:]
    return _tc_mlp(x, partials, global_attr, w1x, w1e, w1g,
                   b1.reshape(1, LATENT), W2, b2.reshape(1, D_OUT))


# speedup vs baseline: 15.0513x; 1.4992x over previous
import functools

import jax
import jax.numpy as jnp
from jax import lax
from jax.experimental import pallas as pl
from jax.experimental.pallas import tpu as pltpu
from jax.experimental.pallas import tpu_sc as plsc

N_NODES = 10000
N_EDGES = 320000
D_EDGE = 16
D_NODE = 128
D_GLOBAL = 128
LATENT = 32
D_OUT = 128

NC = 2
NS = 16
TR = 2
SL = 8
LN = 128
TCOLS = N_EDGES // LN
TC_PER_CORE = TCOLS // NC
CC = 125
CHUNKS = TC_PER_CORE // CC
E_CHUNK = CC * LN
N_NODES_PAD = 10240


def _sc_segment_sum(ea4, ei3):
    mesh = plsc.VectorSubcoreMesh(core_axis_name="c", subcore_axis_name="s")

    @functools.partial(
        pl.kernel,
        mesh=mesh,
        compiler_params=pltpu.CompilerParams(
            use_tc_tiling_on_sc=False, needs_layout_passes=False),
        out_type=jax.ShapeDtypeStruct((NC, D_EDGE, N_NODES_PAD), jnp.float32),
        scratch_types=[
            pltpu.VMEM((2, CC, LN), jnp.float32),
            pltpu.VMEM((2, CC, LN), jnp.int32),
            pltpu.VMEM((N_NODES_PAD,), jnp.float32),
            pltpu.VMEM((N_NODES_PAD,), jnp.float32),
            pltpu.SemaphoreType.DMA,
            pltpu.SemaphoreType.DMA,
        ],
    )
    def seg_sum(ea_hbm, ei_hbm, out_hbm, ebuf, ibuf, acc_a, acc_b,
                sem0, sem1):
        cid = lax.axis_index("c")
        sid = lax.axis_index("s")
        tr = sid // SL
        s = sid % SL
        tc0 = cid * TC_PER_CORE
        sems = (sem0, sem1)

        def start_load(j):
            b = j % 2
            cv = pltpu.async_copy(
                ea_hbm.at[tr, pl.ds(tc0 + j * CC, CC), s, :], ebuf.at[b],
                sems[b])
            ci = pltpu.async_copy(
                ei_hbm.at[pl.ds(tc0 + j * CC, CC), 1, :], ibuf.at[b],
                sems[b])
            return cv, ci

        pend = start_load(0)

        zv = jnp.zeros((16,), jnp.float32)

        def zero(i, carry):
            acc_a[pl.ds(i * 16, 16)] = zv
            acc_b[pl.ds(i * 16, 16)] = zv
            return carry
        lax.fori_loop(0, N_NODES_PAD // 16, zero, 0)

        for j in range(CHUNKS):
            cv, ci = pend
            cv.wait()
            ci.wait()
            if j + 1 < CHUNKS:
                pend = start_load(j + 1)
            b = j % 2

            def row_body(r, carry):
                idxs = [ibuf[b, r, pl.ds(g * 16, 16)] for g in range(SL)]
                vals = [ebuf[b, r, pl.ds(g * 16, 16)] for g in range(SL)]
                for g in range(SL):
                    plsc.addupdate_scatter(acc_a if g % 2 == 0 else acc_b,
                                           [idxs[g]], vals[g])
                return carry
            lax.fori_loop(0, CC, row_body, 0)

        def merge(i, carry):
            sl = pl.ds(i * 16, 16)
            acc_a[sl] = acc_a[sl] + acc_b[sl]
            return carry
        lax.fori_loop(0, N_NODES_PAD // 16, merge, 0)

        pltpu.sync_copy(acc_a, out_hbm.at[cid, sid])

    return seg_sum(ea4, ei3)


BLK = 2048


def _tc_mlp_body(x_ref, p_ref, g_ref, w1x_ref, w1e_ref, w1g_ref, b1_ref,
                 w2_ref, b2_ref, o_ref):
    bias = (jnp.dot(g_ref[...], w1g_ref[...],
                    preferred_element_type=jnp.float32) + b1_ref[...])
    agg_t = p_ref[0] + p_ref[1]
    h = jnp.dot(x_ref[...], w1x_ref[...], preferred_element_type=jnp.float32)
    h = h + lax.dot_general(agg_t, w1e_ref[...], (((0,), (0,)), ((), ())),
                            preferred_element_type=jnp.float32)
    h = jnp.maximum(h + bias, 0.0)
    o_ref[...] = (jnp.dot(h, w2_ref[...], preferred_element_type=jnp.float32)
                  + b2_ref[...])


def _tc_mlp(x, partials, global_attr, w1x, w1e, w1g, b1, w2, b2):
    grid = (N_NODES_PAD // BLK,)
    return pl.pallas_call(
        _tc_mlp_body,
        grid=grid,
        in_specs=[
            pl.BlockSpec((BLK, D_NODE), lambda i: (i, 0)),
            pl.BlockSpec((NC, D_EDGE, BLK), lambda i: (0, 0, i)),
            pl.BlockSpec((1, D_GLOBAL), lambda i: (0, 0)),
            pl.BlockSpec((D_NODE, LATENT), lambda i: (0, 0)),
            pl.BlockSpec((D_EDGE, LATENT), lambda i: (0, 0)),
            pl.BlockSpec((D_GLOBAL, LATENT), lambda i: (0, 0)),
            pl.BlockSpec((1, LATENT), lambda i: (0, 0)),
            pl.BlockSpec((LATENT, D_OUT), lambda i: (0, 0)),
            pl.BlockSpec((1, D_OUT), lambda i: (0, 0)),
        ],
        out_specs=pl.BlockSpec((BLK, D_OUT), lambda i: (i, 0)),
        out_shape=jax.ShapeDtypeStruct((N_NODES, D_OUT), jnp.float32),
    )(x, partials, global_attr, w1x, w1e, w1g, b1, w2, b2)


def kernel(x, edge_index, edge_attr, global_attr, W1, b1, W2, b2):
    ea4 = edge_attr.T.reshape(TR, SL, TCOLS, LN).transpose(0, 2, 1, 3)
    ei3 = edge_index.astype(jnp.int32).reshape(2, TCOLS, LN).transpose(1, 0, 2)
    partials = _sc_segment_sum(ea4, ei3)
    w1x = W1[:D_NODE]
    w1e = W1[D_NODE:D_NODE + D_EDGE]
    w1g = W1[D_NODE + D_EDGE:]
    return _tc_mlp(x, partials, global_attr, w1x, w1e, w1g,
                   b1.reshape(1, LATENT), W2, b2.reshape(1, D_OUT))


# speedup vs baseline: 15.7783x; 1.0483x over previous
import functools

import jax
import jax.numpy as jnp
from jax import lax
from jax.experimental import pallas as pl
from jax.experimental.pallas import tpu as pltpu
from jax.experimental.pallas import tpu_sc as plsc

N_NODES = 10000
N_EDGES = 320000
D_EDGE = 16
D_NODE = 128
D_GLOBAL = 128
LATENT = 32
D_OUT = 128

NC = 2
NS = 16
TR = 2
SL = 8
LN = 128
TCOLS = N_EDGES // LN
TC_PER_CORE = TCOLS // NC
CC = 125
CHUNKS = TC_PER_CORE // CC
E_CHUNK = CC * LN
N_NODES_PAD = 10240


def _sc_segment_sum(ea4, ei3):
    mesh = plsc.VectorSubcoreMesh(core_axis_name="c", subcore_axis_name="s")

    @functools.partial(
        pl.kernel,
        mesh=mesh,
        compiler_params=pltpu.CompilerParams(
            use_tc_tiling_on_sc=False, needs_layout_passes=False),
        out_type=jax.ShapeDtypeStruct((NC, TR, N_NODES_PAD // LN, SL, LN),
                                      jnp.float32),
        scratch_types=[
            pltpu.VMEM((2, CC, LN), jnp.float32),
            pltpu.VMEM((2, CC, LN), jnp.int32),
            pltpu.VMEM((N_NODES_PAD // LN, LN), jnp.float32),
            pltpu.SemaphoreType.DMA,
            pltpu.SemaphoreType.DMA,
        ],
    )
    def seg_sum(ea_hbm, ei_hbm, out_hbm, ebuf, ibuf, acc, sem0, sem1):
        cid = lax.axis_index("c")
        sid = lax.axis_index("s")
        tr = sid // SL
        s = sid % SL
        tc0 = cid * TC_PER_CORE
        sems = (sem0, sem1)

        def start_load(j):
            b = j % 2
            cv = pltpu.async_copy(
                ea_hbm.at[tr, pl.ds(tc0 + j * CC, CC), s, :], ebuf.at[b],
                sems[b])
            ci = pltpu.async_copy(
                ei_hbm.at[pl.ds(tc0 + j * CC, CC), 1, :], ibuf.at[b],
                sems[b])
            return cv, ci

        pend = start_load(0)

        zv = jnp.zeros((16,), jnp.float32)

        def zero(i, carry):
            acc[i // SL, pl.ds((i % SL) * 16, 16)] = zv
            return carry
        lax.fori_loop(0, N_NODES_PAD // 16, zero, 0)

        for j in range(CHUNKS):
            cv, ci = pend
            cv.wait()
            ci.wait()
            if j + 1 < CHUNKS:
                pend = start_load(j + 1)
            b = j % 2

            def row_body(r, carry):
                idxs = [ibuf[b, r, pl.ds(g * 16, 16)] for g in range(SL)]
                vals = [ebuf[b, r, pl.ds(g * 16, 16)] for g in range(SL)]
                for g in range(SL):
                    plsc.addupdate_scatter(
                        acc, [idxs[g] >> 7, idxs[g] & 127], vals[g])
                return carry
            lax.fori_loop(0, CC, row_body, 0)

        pltpu.sync_copy(acc, out_hbm.at[cid, tr, :, s, :])

    return seg_sum(ea4, ei3)


BLK = 2048


def _tc_mlp1_body(x_ref, g_ref, w1x_ref, w1g_ref, b1_ref, h_ref):
    bias = (jnp.dot(g_ref[...], w1g_ref[...],
                    preferred_element_type=jnp.float32) + b1_ref[...])
    h_ref[...] = (jnp.dot(x_ref[...], w1x_ref[...],
                          preferred_element_type=jnp.float32) + bias)


def _tc_mlp1(x, global_attr, w1x, w1g, b1):
    grid = (N_NODES_PAD // BLK,)
    return pl.pallas_call(
        _tc_mlp1_body,
        grid=grid,
        in_specs=[
            pl.BlockSpec((BLK, D_NODE), lambda i: (i, 0)),
            pl.BlockSpec((1, D_GLOBAL), lambda i: (0, 0)),
            pl.BlockSpec((D_NODE, LATENT), lambda i: (0, 0)),
            pl.BlockSpec((D_GLOBAL, LATENT), lambda i: (0, 0)),
            pl.BlockSpec((1, LATENT), lambda i: (0, 0)),
        ],
        out_specs=pl.BlockSpec((BLK, LATENT), lambda i: (i, 0)),
        out_shape=jax.ShapeDtypeStruct((N_NODES, LATENT), jnp.float32),
    )(x, global_attr, w1x, w1g, b1)


def _tc_mlp2_body(h_ref, p_ref, w1e_ref, w2_ref, b2_ref, o_ref):
    agg_t = p_ref[0] + p_ref[1]
    h = h_ref[...] + lax.dot_general(
        agg_t, w1e_ref[...], (((0,), (0,)), ((), ())),
        preferred_element_type=jnp.float32)
    h = jnp.maximum(h, 0.0)
    o_ref[...] = (jnp.dot(h, w2_ref[...], preferred_element_type=jnp.float32)
                  + b2_ref[...])


def _tc_mlp2(h1, partials, w1e, w2, b2):
    grid = (N_NODES_PAD // BLK,)
    return pl.pallas_call(
        _tc_mlp2_body,
        grid=grid,
        in_specs=[
            pl.BlockSpec((BLK, LATENT), lambda i: (i, 0)),
            pl.BlockSpec((NC, D_EDGE, BLK), lambda i: (0, 0, i)),
            pl.BlockSpec((D_EDGE, LATENT), lambda i: (0, 0)),
            pl.BlockSpec((LATENT, D_OUT), lambda i: (0, 0)),
            pl.BlockSpec((1, D_OUT), lambda i: (0, 0)),
        ],
        out_specs=pl.BlockSpec((BLK, D_OUT), lambda i: (i, 0)),
        out_shape=jax.ShapeDtypeStruct((N_NODES, D_OUT), jnp.float32),
    )(h1, partials, w1e, w2, b2)


def kernel(x, edge_index, edge_attr, global_attr, W1, b1, W2, b2):
    ea4 = edge_attr.T.reshape(TR, SL, TCOLS, LN).transpose(0, 2, 1, 3)
    ei3 = edge_index.astype(jnp.int32).reshape(2, TCOLS, LN).transpose(1, 0, 2)
    p5 = _sc_segment_sum(ea4, ei3)
    partials = p5.transpose(0, 1, 3, 2, 4).reshape(NC, D_EDGE, N_NODES_PAD)
    w1x = W1[:D_NODE]
    w1e = W1[D_NODE:D_NODE + D_EDGE]
    w1g = W1[D_NODE + D_EDGE:]
    h1 = _tc_mlp1(x, global_attr, w1x, w1g, b1.reshape(1, LATENT))
    return _tc_mlp2(h1, partials, w1e, W2, b2.reshape(1, D_OUT))


# speedup vs baseline: 15.7813x; 1.0002x over previous
import functools

import jax
import jax.numpy as jnp
from jax import lax
from jax.experimental import pallas as pl
from jax.experimental.pallas import tpu as pltpu
from jax.experimental.pallas import tpu_sc as plsc

N_NODES = 10000
N_EDGES = 320000
D_EDGE = 16
D_NODE = 128
D_GLOBAL = 128
LATENT = 32
D_OUT = 128

NC = 2
NS = 16
TR = 2
SL = 8
LN = 128
TCOLS = N_EDGES // LN
TC_PER_CORE = TCOLS // NC
CC = 25
CHUNKS = TC_PER_CORE // CC // 2
N_NODES_PAD = 10240


def _sc_segment_sum(ea4, ei3):
    mesh = plsc.VectorSubcoreMesh(core_axis_name="c", subcore_axis_name="s")

    @functools.partial(
        pl.kernel,
        mesh=mesh,
        compiler_params=pltpu.CompilerParams(
            use_tc_tiling_on_sc=False, needs_layout_passes=False),
        out_type=jax.ShapeDtypeStruct((NC * 2, TR, N_NODES_PAD // LN, SL, LN),
                                      jnp.float32),
        scratch_types=[
            pltpu.VMEM((2, CC, 2, LN), jnp.float32),
            pltpu.VMEM((2, CC, LN), jnp.int32),
            pltpu.VMEM((N_NODES_PAD // LN, LN), jnp.float32),
            pltpu.VMEM((N_NODES_PAD // LN, LN), jnp.float32),
            pltpu.SemaphoreType.DMA,
            pltpu.SemaphoreType.DMA,
        ],
    )
    def seg_sum(ea_hbm, ei_hbm, out_hbm, ebuf, ibuf, acc_a, acc_b,
                sem0, sem1):
        cid = lax.axis_index("c")
        sid = lax.axis_index("s")
        half = sid // SL
        fp = sid % SL
        tr = (2 * fp) // SL
        s0 = (2 * fp) % SL
        tc0 = cid * TC_PER_CORE
        sems = (sem0, sem1)

        def start_load(j):
            b = j % 2
            tc = tc0 + (2 * j + half) * CC
            cv = pltpu.async_copy(
                ea_hbm.at[tr, pl.ds(tc, CC), pl.ds(s0, 2), :], ebuf.at[b],
                sems[b])
            ci = pltpu.async_copy(
                ei_hbm.at[pl.ds(tc, CC), 1, :], ibuf.at[b],
                sems[b])
            return cv, ci

        pend = start_load(0)

        zv = jnp.zeros((16,), jnp.float32)

        def zero(i, carry):
            acc_a[i // SL, pl.ds((i % SL) * 16, 16)] = zv
            acc_b[i // SL, pl.ds((i % SL) * 16, 16)] = zv
            return carry
        lax.fori_loop(0, N_NODES_PAD // 16, zero, 0)

        for j in range(CHUNKS):
            cv, ci = pend
            cv.wait()
            ci.wait()
            if j + 1 < CHUNKS:
                pend = start_load(j + 1)
            b = j % 2

            def row_body(r, carry):
                idxs = [ibuf[b, r, pl.ds(g * 16, 16)] for g in range(SL)]
                va = [ebuf[b, r, 0, pl.ds(g * 16, 16)] for g in range(SL)]
                vb = [ebuf[b, r, 1, pl.ds(g * 16, 16)] for g in range(SL)]
                his = [idx >> 7 for idx in idxs]
                los = [idx & 127 for idx in idxs]
                for g in range(SL):
                    plsc.addupdate_scatter(acc_a, [his[g], los[g]], va[g])
                    plsc.addupdate_scatter(acc_b, [his[g], los[g]], vb[g])
                return carry
            lax.fori_loop(0, CC, row_body, 0)

        part = cid * 2 + half
        pltpu.sync_copy(acc_a, out_hbm.at[part, tr, :, s0, :])
        pltpu.sync_copy(acc_b, out_hbm.at[part, tr, :, s0 + 1, :])

    return seg_sum(ea4, ei3)


BLK = 2048


def _tc_mlp1_body(x_ref, g_ref, w1x_ref, w1g_ref, b1_ref, h_ref):
    bias = (jnp.dot(g_ref[...], w1g_ref[...],
                    preferred_element_type=jnp.float32) + b1_ref[...])
    h_ref[...] = (jnp.dot(x_ref[...], w1x_ref[...],
                          preferred_element_type=jnp.float32) + bias)


def _tc_mlp1(x, global_attr, w1x, w1g, b1):
    grid = (N_NODES_PAD // BLK,)
    return pl.pallas_call(
        _tc_mlp1_body,
        grid=grid,
        in_specs=[
            pl.BlockSpec((BLK, D_NODE), lambda i: (i, 0)),
            pl.BlockSpec((1, D_GLOBAL), lambda i: (0, 0)),
            pl.BlockSpec((D_NODE, LATENT), lambda i: (0, 0)),
            pl.BlockSpec((D_GLOBAL, LATENT), lambda i: (0, 0)),
            pl.BlockSpec((1, LATENT), lambda i: (0, 0)),
        ],
        out_specs=pl.BlockSpec((BLK, LATENT), lambda i: (i, 0)),
        out_shape=jax.ShapeDtypeStruct((N_NODES, LATENT), jnp.float32),
    )(x, global_attr, w1x, w1g, b1)


def _tc_mlp2_body(h_ref, p_ref, w1e_ref, w2_ref, b2_ref, o_ref):
    agg_t = ((p_ref[0] + p_ref[1]) + (p_ref[2] + p_ref[3]))
    h = h_ref[...] + lax.dot_general(
        agg_t, w1e_ref[...], (((0,), (0,)), ((), ())),
        preferred_element_type=jnp.float32)
    h = jnp.maximum(h, 0.0)
    o_ref[...] = (jnp.dot(h, w2_ref[...], preferred_element_type=jnp.float32)
                  + b2_ref[...])


def _tc_mlp2(h1, partials, w1e, w2, b2):
    grid = (N_NODES_PAD // BLK,)
    return pl.pallas_call(
        _tc_mlp2_body,
        grid=grid,
        in_specs=[
            pl.BlockSpec((BLK, LATENT), lambda i: (i, 0)),
            pl.BlockSpec((NC * 2, D_EDGE, BLK), lambda i: (0, 0, i)),
            pl.BlockSpec((D_EDGE, LATENT), lambda i: (0, 0)),
            pl.BlockSpec((LATENT, D_OUT), lambda i: (0, 0)),
            pl.BlockSpec((1, D_OUT), lambda i: (0, 0)),
        ],
        out_specs=pl.BlockSpec((BLK, D_OUT), lambda i: (i, 0)),
        out_shape=jax.ShapeDtypeStruct((N_NODES, D_OUT), jnp.float32),
    )(h1, partials, w1e, w2, b2)


def kernel(x, edge_index, edge_attr, global_attr, W1, b1, W2, b2):
    ea4 = edge_attr.T.reshape(TR, SL, TCOLS, LN).transpose(0, 2, 1, 3)
    ei3 = edge_index.astype(jnp.int32).reshape(2, TCOLS, LN).transpose(1, 0, 2)
    p5 = _sc_segment_sum(ea4, ei3)
    partials = p5.transpose(0, 1, 3, 2, 4).reshape(NC * 2, D_EDGE, N_NODES_PAD)
    w1x = W1[:D_NODE]
    w1e = W1[D_NODE:D_NODE + D_EDGE]
    w1g = W1[D_NODE + D_EDGE:]
    h1 = _tc_mlp1(x, global_attr, w1x, w1g, b1.reshape(1, LATENT))
    return _tc_mlp2(h1, partials, w1e, W2, b2.reshape(1, D_OUT))
